# Initial kernel scaffold; baseline (speedup 1.0000x reference)
import jax
import jax.numpy as jnp
from jax.experimental import pallas as pl


def kernel(hidden_states, W):
    raise NotImplementedError("write your pallas kernel here")


# speedup vs baseline: 1.0400x; 1.0400x over previous
import functools

import jax
import jax.numpy as jnp
from jax.experimental import pallas as pl
from jax.experimental.pallas import tpu as pltpu

TOPK = 8
NEXP = 64
BLK = 512


def _gate_block(x_ref, wt_ref, w_out_ref, i_out_ref):
    x = x_ref[...]
    wt = wt_ref[...]
    logits = jax.lax.dot_general(
        x, wt, (((1,), (0,)), ((), ())),
        preferred_element_type=jnp.float32,
        precision=jax.lax.Precision.DEFAULT,
    )
    m = jnp.max(logits, axis=1, keepdims=True)
    e = jnp.exp(logits - m)
    colid = jax.lax.broadcasted_iota(jnp.int32, e.shape, 1)
    cur = e
    vals = []
    idxs = []
    for _ in range(TOPK):
        mk = jnp.max(cur, axis=1, keepdims=True)
        hit = cur == mk
        ik = jnp.min(jnp.where(hit, colid, NEXP), axis=1, keepdims=True)
        vals.append(mk)
        idxs.append(ik)
        cur = jnp.where(colid == ik, -1.0, cur)
    w = jnp.concatenate(vals, axis=1)
    w = w / (jnp.sum(w, axis=1, keepdims=True) + 1e-20)
    w_out_ref[...] = w
    i_out_ref[...] = jnp.concatenate(idxs, axis=1)


@jax.jit
def kernel(hidden_states, W):
    b, s, h = hidden_states.shape
    n = b * s
    x = hidden_states.reshape(n, h)
    wt = W.astype(jnp.float32).T
    grid = (n // BLK,)
    w_out, i_out = pl.pallas_call(
        _gate_block,
        grid=grid,
        in_specs=[
            pl.BlockSpec((BLK, h), lambda i: (i, 0)),
            pl.BlockSpec((h, NEXP), lambda i: (0, 0)),
        ],
        out_specs=[
            pl.BlockSpec((BLK, TOPK), lambda i: (i, 0)),
            pl.BlockSpec((BLK, TOPK), lambda i: (i, 0)),
        ],
        out_shape=[
            jax.ShapeDtypeStruct((n, TOPK), jnp.float32),
            jax.ShapeDtypeStruct((n, TOPK), jnp.int32),
        ],
    )(x, wt)
    return w_out, i_out


# speedup vs baseline: 1.2806x; 1.2313x over previous
import functools

import jax
import jax.numpy as jnp
from jax.experimental import pallas as pl
from jax.experimental.pallas import tpu as pltpu

TOPK = 8
NEXP = 64
BLK = 512


CHUNK = 128


def _topk_chunk(e):
    colid = jax.lax.broadcasted_iota(jnp.int32, e.shape, 1).astype(jnp.float32)
    cur = e
    vals = []
    idxs = []
    for _ in range(TOPK):
        mk = jnp.max(cur, axis=1, keepdims=True)
        ik = jnp.min(jnp.where(cur == mk, colid, float(NEXP)), axis=1,
                     keepdims=True)
        vals.append(mk)
        idxs.append(ik)
        cur = jnp.where(colid == ik, -1.0, cur)
    w = jnp.concatenate(vals, axis=1)
    w = w / (jnp.sum(w, axis=1, keepdims=True) + 1e-20)
    return w, jnp.concatenate(idxs, axis=1).astype(jnp.int32)


def _gate_block(x_ref, wt_ref, w_out_ref, i_out_ref):
    x = x_ref[...]
    wt = wt_ref[...]
    logits = jax.lax.dot_general(
        x, wt, (((1,), (0,)), ((), ())),
        preferred_element_type=jnp.float32,
        precision=jax.lax.Precision.DEFAULT,
    )
    m = jnp.max(logits, axis=1, keepdims=True)
    e = jnp.exp(logits - m)
    for c in range(BLK // CHUNK):
        lo, hi = c * CHUNK, (c + 1) * CHUNK
        w, ik = _topk_chunk(jax.lax.slice(e, (lo, 0), (hi, NEXP)))
        w_out_ref[lo:hi, :] = w
        i_out_ref[lo:hi, :] = ik


@jax.jit
def kernel(hidden_states, W):
    b, s, h = hidden_states.shape
    n = b * s
    x = hidden_states.reshape(n, h)
    wt = W.astype(jnp.float32).T
    grid = (n // BLK,)
    w_out, i_out = pl.pallas_call(
        _gate_block,
        grid=grid,
        in_specs=[
            pl.BlockSpec((BLK, h), lambda i: (i, 0)),
            pl.BlockSpec((h, NEXP), lambda i: (0, 0)),
        ],
        out_specs=[
            pl.BlockSpec((BLK, TOPK), lambda i: (i, 0)),
            pl.BlockSpec((BLK, TOPK), lambda i: (i, 0)),
        ],
        out_shape=[
            jax.ShapeDtypeStruct((n, TOPK), jnp.float32),
            jax.ShapeDtypeStruct((n, TOPK), jnp.int32),
        ],
    )(x, wt)
    return w_out, i_out


# speedup vs baseline: 1.2837x; 1.0025x over previous
import functools

import jax
import jax.numpy as jnp
from jax.experimental import pallas as pl
from jax.experimental.pallas import tpu as pltpu

TOPK = 8
NEXP = 64
BLK = 512


CHUNK = 128


def _topk_chunk(e):
    colid = jax.lax.broadcasted_iota(jnp.int32, e.shape, 1).astype(jnp.float32)
    cur = e
    vals = []
    idxs = []
    for _ in range(TOPK):
        mk = jnp.max(cur, axis=1, keepdims=True)
        ik = jnp.min(jnp.where(cur == mk, colid, float(NEXP)), axis=1,
                     keepdims=True)
        vals.append(mk)
        idxs.append(ik)
        cur = jnp.where(colid == ik, -1.0, cur)
    w = jnp.concatenate(vals, axis=1)
    w = w / (jnp.sum(w, axis=1, keepdims=True) + 1e-20)
    return w, jnp.concatenate(idxs, axis=1).astype(jnp.int32)


def _gate_block(x_ref, wt_ref, w_out_ref, i_out_ref):
    x = x_ref[...]
    wt = wt_ref[...]
    logits = jax.lax.dot_general(
        x, wt, (((1,), (0,)), ((), ())),
        preferred_element_type=jnp.float32,
        precision=jax.lax.Precision.DEFAULT,
    )
    m = jnp.max(logits, axis=1, keepdims=True)
    e = jnp.exp(logits - m)
    for c in range(BLK // CHUNK):
        lo, hi = c * CHUNK, (c + 1) * CHUNK
        w, ik = _topk_chunk(jax.lax.slice(e, (lo, 0), (hi, NEXP)))
        w_out_ref[lo:hi, :] = w
        i_out_ref[lo:hi, :] = ik


@jax.jit
def kernel(hidden_states, W):
    b, s, h = hidden_states.shape
    n = b * s
    x = hidden_states.reshape(n, h)
    wt = W.astype(jnp.float32).T
    grid = (n // BLK,)
    w_out, i_out = pl.pallas_call(
        _gate_block,
        grid=grid,
        in_specs=[
            pl.BlockSpec((BLK, h), lambda i: (i, 0)),
            pl.BlockSpec((h, NEXP), lambda i: (0, 0)),
        ],
        out_specs=[
            pl.BlockSpec((BLK, TOPK), lambda i: (i, 0)),
            pl.BlockSpec((BLK, TOPK), lambda i: (i, 0)),
        ],
        out_shape=[
            jax.ShapeDtypeStruct((n, TOPK), jnp.float32),
            jax.ShapeDtypeStruct((n, TOPK), jnp.int32),
        ],
        compiler_params=pltpu.CompilerParams(
            dimension_semantics=("parallel",),
        ),
    )(x, wt)
    return w_out, i_out
